# baseline (speedup 1.0000x reference)
import functools

import jax
import jax.numpy as jnp
import numpy as np
from jax.experimental import pallas as pl
from jax.experimental.pallas import tpu as pltpu
from jax.experimental.pallas import tpu_sc as plsc

THRESH = np.float32(0.7)
C = 19
ROWS = 256
NW = 32
PER_W = 32768
CH_E = 8192


def _stage1_body(score_ref, target_ref, pred_ref, loss_ref):
    t = target_ref[0]
    m = score_ref[0, 0]
    for c in range(1, C):
        m = jnp.maximum(m, score_ref[0, c])
    se = jnp.zeros_like(m)
    st = jnp.zeros_like(m)
    for c in range(C):
        s = score_ref[0, c]
        se = se + jnp.exp(s - m)
        st = jnp.where(t == c, s, st)
    pred_ref[...] = jnp.exp(st - m) / se
    loss_ref[...] = -st


def _sc_select(pred_hbm, loss_hbm, out_hbm, pred_v, loss_v, res_v):
    w = jax.lax.axis_index("s") * 2 + jax.lax.axis_index("c")
    base = w * PER_W
    zero = jnp.zeros((16,), jnp.float32)

    def outer(j, accs):
        off = base + j * CH_E
        pltpu.sync_copy(pred_hbm.at[pl.ds(off, CH_E)], pred_v)
        pltpu.sync_copy(loss_hbm.at[pl.ds(off, CH_E)], loss_v)

        def inner(ti, accs2):
            c_le, c_lt, s_lt = accs2
            for b in range(8):
                sl = pl.ds(ti * 128 + b * 16, 16)
                x = pred_v[sl]
                l = loss_v[sl]
                lt = jnp.where(x < THRESH, 1.0, 0.0)
                c_le = c_le + jnp.where(x <= THRESH, 1.0, 0.0)
                c_lt = c_lt + lt
                s_lt = s_lt + l * lt
            return (c_le, c_lt, s_lt)

        return jax.lax.fori_loop(0, CH_E // 128, inner, accs)

    c_le, c_lt, s_lt = jax.lax.fori_loop(
        0, PER_W // CH_E, outer, (zero, zero, zero))
    res_v[pl.ds(0, 16)] = c_le
    res_v[pl.ds(16, 16)] = c_lt
    res_v[pl.ds(32, 16)] = s_lt
    pltpu.sync_copy(res_v, out_hbm.at[w])


@jax.jit
def kernel(score, target):
    B, Cc, H, W = score.shape
    n_chunks = H // ROWS
    steps = B * n_chunks
    k = int(0.7 * H * W)

    pred, loss = pl.pallas_call(
        _stage1_body,
        grid=(steps,),
        in_specs=[
            pl.BlockSpec((1, Cc, ROWS, W),
                         lambda i: (i // n_chunks, 0, i % n_chunks, 0)),
            pl.BlockSpec((1, ROWS, W),
                         lambda i: (i // n_chunks, i % n_chunks, 0)),
        ],
        out_specs=[
            pl.BlockSpec((ROWS, W), lambda i: (i, 0)),
            pl.BlockSpec((ROWS, W), lambda i: (i, 0)),
        ],
        out_shape=[
            jax.ShapeDtypeStruct((steps * ROWS, W), jnp.float32),
            jax.ShapeDtypeStruct((steps * ROWS, W), jnp.float32),
        ],
        compiler_params=pltpu.CompilerParams(
            dimension_semantics=("arbitrary",),
        ),
    )(score, target)

    pred1 = pred.reshape(-1)
    loss1 = loss.reshape(-1)

    mesh = plsc.VectorSubcoreMesh(core_axis_name="c", subcore_axis_name="s")
    sc_sel = functools.partial(
        pl.kernel,
        mesh=mesh,
        out_type=jax.ShapeDtypeStruct((NW, 48), jnp.float32),
        scratch_types=[
            pltpu.VMEM((CH_E,), jnp.float32),
            pltpu.VMEM((CH_E,), jnp.float32),
            pltpu.VMEM((48,), jnp.float32),
        ],
    )(_sc_select)

    parts = sc_sel(pred1, loss1)
    c07 = jnp.sum(parts[:, 0:16])
    kc = jnp.sum(parts[:, 16:32])
    ks = jnp.sum(parts[:, 32:48])

    def fast(_):
        return ks / jnp.maximum(kc, 1.0)

    def slow(_):
        ps = jnp.sort(pred1)
        thr = jnp.maximum(ps[k], THRESH)
        keep = (pred1 < thr).astype(jnp.float32)
        return jnp.sum(loss1 * keep) / jnp.maximum(jnp.sum(keep), 1.0)

    return jax.lax.cond(c07 >= k + 1, fast, slow, None)


# speedup vs baseline: 2.3839x; 2.3839x over previous
import jax
import jax.numpy as jnp
import numpy as np
from jax.experimental import pallas as pl
from jax.experimental.pallas import tpu as pltpu

THRESH = np.float32(0.7)
C = 19
ROWS = 256


def _body(k, steps, score_ref, target_ref, out_ref, pred_buf, loss_buf,
          acc_buf):
    i = pl.program_id(0)

    def compute_chunk():
        t = target_ref[0]
        m = score_ref[0, 0]
        for c in range(1, C):
            m = jnp.maximum(m, score_ref[0, c])
        se = jnp.zeros_like(m)
        st = jnp.zeros_like(m)
        for c in range(C):
            s = score_ref[0, c]
            se = se + jnp.exp(s - m)
            st = jnp.where(t == c, s, st)
        pred = jnp.exp(st - m) / se
        loss = -st
        rows = pl.ds(i * ROWS, ROWS)
        pred_buf[rows, :] = pred
        loss_buf[rows, :] = loss
        le = (pred <= THRESH).astype(jnp.float32)
        lt = (pred < THRESH).astype(jnp.float32)
        first = (i == 0)
        acc_buf[0] = jnp.where(first, le, acc_buf[0] + le)
        acc_buf[1] = jnp.where(first, lt, acc_buf[1] + lt)
        sl = loss * lt
        acc_buf[2] = jnp.where(first, sl, acc_buf[2] + sl)

    compute_chunk()

    @pl.when(i == steps - 1)
    def epilogue():
        c07 = jnp.sum(acc_buf[0])

        def fast(_):
            return acc_buf[2].sum() / jnp.maximum(acc_buf[1].sum(), 1.0)

        def slow(_):
            x = pred_buf[...]
            xb = jax.lax.bitcast_convert_type(x, jnp.int32)

            def bisect(_, carry):
                lo, hi = carry
                mid = (lo + hi) // 2
                c = jnp.sum((xb <= mid).astype(jnp.int32))
                take_hi = c >= k + 1
                return (jnp.where(take_hi, lo, mid),
                        jnp.where(take_hi, mid, hi))

            _, hi = jax.lax.fori_loop(
                0, 31, bisect, (jnp.int32(-1), jnp.int32(0x3F800000)))
            v = jax.lax.bitcast_convert_type(hi, jnp.float32)
            thr = jnp.maximum(v, THRESH)
            keep = (x < thr).astype(jnp.float32)
            ks = jnp.sum(loss_buf[...] * keep)
            kc = jnp.sum(keep)
            return ks / jnp.maximum(kc, 1.0)

        out_ref[0, 0] = jax.lax.cond(c07 >= k + 1, fast, slow, None)


@jax.jit
def kernel(score, target):
    B, Cc, H, W = score.shape
    n_chunks = H // ROWS
    steps = B * n_chunks
    k = int(0.7 * H * W)

    def score_map(i):
        return (i // n_chunks, 0, i % n_chunks, 0)

    def target_map(i):
        return (i // n_chunks, i % n_chunks, 0)

    body = lambda *refs: _body(k, steps, *refs)

    out = pl.pallas_call(
        body,
        grid=(steps,),
        in_specs=[
            pl.BlockSpec((1, Cc, ROWS, W), score_map),
            pl.BlockSpec((1, ROWS, W), target_map),
        ],
        out_specs=pl.BlockSpec(memory_space=pltpu.SMEM),
        out_shape=jax.ShapeDtypeStruct((1, 1), jnp.float32),
        scratch_shapes=[
            pltpu.VMEM((steps * ROWS, W), jnp.float32),
            pltpu.VMEM((steps * ROWS, W), jnp.float32),
            pltpu.VMEM((3, ROWS, W), jnp.float32),
        ],
        compiler_params=pltpu.CompilerParams(
            dimension_semantics=("arbitrary",),
        ),
    )(score, target)
    return out[0, 0]
